# baseline (speedup 1.0000x reference)
import jax
import jax.numpy as jnp
from jax import lax
from jax.experimental import pallas as pl
from jax.experimental.pallas import tpu as pltpu
from jax.experimental.pallas import tpu_sc as plsc

N = 10000
E = 160000
NC, NS = 2, 16
KB = 128
NB = 40
EW = KB * NB
EP = EW * NC * NS
NROW = 10240
RPT = NROW // NS
DUMP = N

P1, P2, P3, P4 = 112, 64, 32, 16
RB = 1000


def _deg_body(dst_hbm, deg_out, dst_v, degtab, sumv, shared):
    cid = lax.axis_index("c")
    sid = lax.axis_index("s")
    pltpu.sync_copy(dst_hbm.at[cid, sid], dst_v)

    def _zero(i, _):
        degtab[pl.ds(i * 16, 16)] = jnp.zeros((16,), jnp.float32)
        return 0
    lax.fori_loop(0, NROW // 16, _zero, 0)

    ones = jnp.ones((16,), jnp.float32)

    def _hist(i, _):
        j = i // (KB // 16)
        c = i % (KB // 16)
        idx = dst_v[j, pl.ds(c * 16, 16)]
        plsc.addupdate_scatter(degtab, [idx], ones)
        return 0
    lax.fori_loop(0, EW // 16, _hist, 0)

    pltpu.sync_copy(degtab, shared.at[sid])
    plsc.subcore_barrier()
    pltpu.sync_copy(shared.at[:, pl.ds(sid * RPT, RPT)], sumv)

    def _red(ci, _):
        a = jnp.zeros((16,), jnp.float32)
        for r in range(NS):
            a = a + sumv[r, pl.ds(ci * 16, 16)]
        degtab[pl.ds(ci * 16, 16)] = a
        return 0
    lax.fori_loop(0, RPT // 16, _red, 0)
    pltpu.sync_copy(degtab.at[pl.ds(0, RPT)], deg_out.at[cid, pl.ds(sid * RPT, RPT)])


_deg_kernel = pl.kernel(
    _deg_body,
    out_type=jax.ShapeDtypeStruct((NC, NROW), jnp.float32),
    mesh=plsc.VectorSubcoreMesh(core_axis_name="c", subcore_axis_name="s"),
    compiler_params=pltpu.CompilerParams(needs_layout_passes=False),
    scratch_types=[
        pltpu.VMEM((NB, KB), jnp.int32),
        pltpu.VMEM((NROW,), jnp.float32),
        pltpu.VMEM((NS, RPT), jnp.float32),
        pltpu.VMEM_SHARED((NS, NROW), jnp.float32),
    ],
)


def _make_agg_body(nbuf, kb, d):
    nb = EP // NS // kb

    def _agg_body(yw_hbm, src_hbm, dst_hbm, zrows_hbm, acc_out,
                  src_v, dst_v, *rest):
        rows = rest[:nbuf]
        gsems = rest[nbuf:2 * nbuf]
        ssems = rest[2 * nbuf:3 * nbuf]
        acc_sp = rest[3 * nbuf]
        cid = lax.axis_index("c")
        sid = lax.axis_index("s")

        @pl.when(cid == 0)
        def _work():
            pltpu.sync_copy(zrows_hbm, rows[0])
            for r in range(RPT // kb):
                pltpu.sync_copy(rows[0], acc_sp.at[pl.ds(sid * RPT + r * kb, kb)])
            plsc.subcore_barrier()

            pltpu.sync_copy(src_hbm.at[sid], src_v)
            pltpu.sync_copy(dst_hbm.at[sid], dst_v)

            def _gather(i, b, sem):
                return pltpu.make_async_copy(yw_hbm.at[src_v.at[i]], rows[b], sem)

            def _scatter(i, b, sem):
                return pltpu.make_async_copy(rows[b], acc_sp.at[dst_v.at[i]], sem)

            for b in range(nbuf):
                _gather(b, b, gsems[b]).start()

            nw = jnp.where(cid == 0, nb // nbuf, 0)

            def _wave(w, _):
                i0 = w * nbuf
                for b in range(nbuf):
                    _gather(i0 + b, b, gsems[b]).wait()
                    _scatter(i0 + b, b, ssems[b]).start(add=True)
                for b in range(nbuf):
                    _scatter(i0 + b, b, ssems[b]).wait()
                    _gather(i0 + nbuf + b, b, gsems[b]).start()
                return 0
            lax.fori_loop(0, nw - 1, _wave, 0)
            i0 = (nw - 1) * nbuf
            for b in range(nbuf):
                _gather(i0 + b, b, gsems[b]).wait()
                _scatter(i0 + b, b, ssems[b]).start(add=True)
            for b in range(nbuf):
                _scatter(i0 + b, b, ssems[b]).wait()
            plsc.subcore_barrier()
            pltpu.sync_copy(acc_sp.at[pl.ds(sid * RPT, RPT)],
                            acc_out.at[pl.ds(sid * RPT, RPT)])
    return _agg_body


def _make_agg(d, nbuf, kb):
    nb = EP // NS // kb
    return pl.kernel(
        _make_agg_body(nbuf, kb, d),
        out_type=jax.ShapeDtypeStruct((NROW, d), jnp.float32),
        mesh=plsc.VectorSubcoreMesh(core_axis_name="c", subcore_axis_name="s"),
        compiler_params=pltpu.CompilerParams(use_tc_tiling_on_sc=False),
        scratch_types=(
            [pltpu.VMEM((nb, kb), jnp.int32),
             pltpu.VMEM((nb, kb), jnp.int32)]
            + [pltpu.VMEM((kb, d), jnp.float32) for _ in range(nbuf)]
            + [pltpu.SemaphoreType.DMA for _ in range(2 * nbuf)]
            + [pltpu.VMEM_SHARED((NROW, d), jnp.float32)]
        ),
    )


def _pre_body(x_ref, w_ref, degt_ref, yw_ref, dinv_ref):
    deg = degt_ref[:, 0:1] + degt_ref[:, 1:2] + 1.0
    dv = lax.rsqrt(deg)
    xw = jnp.dot(x_ref[...], w_ref[...], preferred_element_type=jnp.float32)
    yw_ref[...] = dv * xw
    dinv_ref[...] = dv


def _tc_pre(x, w1p, degt):
    return pl.pallas_call(
        _pre_body,
        grid=(N // RB,),
        in_specs=[
            pl.BlockSpec((RB, x.shape[1]), lambda i: (i, 0)),
            pl.BlockSpec((w1p.shape[0], w1p.shape[1]), lambda i: (0, 0)),
            pl.BlockSpec((RB, 2), lambda i: (i, 0)),
        ],
        out_specs=[
            pl.BlockSpec((RB, w1p.shape[1]), lambda i: (i, 0)),
            pl.BlockSpec((RB, 1), lambda i: (i, 0)),
        ],
        out_shape=[
            jax.ShapeDtypeStruct((N, w1p.shape[1]), jnp.float32),
            jax.ShapeDtypeStruct((N, 1), jnp.float32),
        ],
    )(x, w1p, degt)


def _mid_body(acc_ref, yw_ref, dinv_ref, b_ref, w_ref, out_ref):
    dv = dinv_ref[...]
    h = dv * (acc_ref[...] + yw_ref[...]) + b_ref[...]
    h = jnp.maximum(h, 0.0)
    out_ref[...] = dv * jnp.dot(h, w_ref[...], preferred_element_type=jnp.float32)


def _tc_mid(acc, yw, dinv, bp, wp):
    din, dout = wp.shape
    return pl.pallas_call(
        _mid_body,
        grid=(N // RB,),
        in_specs=[
            pl.BlockSpec((RB, din), lambda i: (i, 0)),
            pl.BlockSpec((RB, din), lambda i: (i, 0)),
            pl.BlockSpec((RB, 1), lambda i: (i, 0)),
            pl.BlockSpec((1, din), lambda i: (0, 0)),
            pl.BlockSpec((din, dout), lambda i: (0, 0)),
        ],
        out_specs=pl.BlockSpec((RB, dout), lambda i: (i, 0)),
        out_shape=jax.ShapeDtypeStruct((N, dout), jnp.float32),
    )(acc, yw, dinv, bp, wp)


def _post_body(acc_ref, yw_ref, dinv_ref, b_ref, out_ref):
    dv = dinv_ref[...]
    out_ref[...] = dv * (acc_ref[...] + yw_ref[...]) + b_ref[...]


def _tc_post(acc, yw, dinv, bp):
    din = yw.shape[1]
    return pl.pallas_call(
        _post_body,
        grid=(N // RB,),
        in_specs=[
            pl.BlockSpec((RB, din), lambda i: (i, 0)),
            pl.BlockSpec((RB, din), lambda i: (i, 0)),
            pl.BlockSpec((RB, 1), lambda i: (i, 0)),
            pl.BlockSpec((1, din), lambda i: (0, 0)),
        ],
        out_specs=pl.BlockSpec((RB, din), lambda i: (i, 0)),
        out_shape=jax.ShapeDtypeStruct((N, din), jnp.float32),
    )(acc, yw, dinv, bp)


def _pad2(a, rows, cols):
    return jnp.pad(a, ((0, rows - a.shape[0]), (0, cols - a.shape[1])))


_CFG1 = (4, 64)
_CFG2 = (8, 128)
_CFG3 = (8, 128)
_CFG4 = (8, 128)


def kernel(x, edge_index, W1, b1, W2, b2, W3, b3, W4, b4):
    src = edge_index[0].astype(jnp.int32)
    dst = edge_index[1].astype(jnp.int32)
    pad = EP - E
    srcp = jnp.concatenate([src, jnp.zeros((pad,), jnp.int32)])
    dump = DUMP + (jnp.arange(pad, dtype=jnp.int32) % (NROW - N))
    dstp = jnp.concatenate([dst, dump])
    src_r = srcp.reshape(NC, NS, NB, KB)
    dst_r = dstp.reshape(NC, NS, NB, KB)
    s64 = srcp.reshape(NS, EP // NS // 64, 64)
    d64 = dstp.reshape(NS, EP // NS // 64, 64)
    s128 = srcp.reshape(NS, EP // NS // 128, 128)
    d128 = dstp.reshape(NS, EP // NS // 128, 128)

    w1p = _pad2(W1, 256, P1)
    w2p = _pad2(W2, P1, P2)
    w3p = _pad2(W3, P2, P3)
    w4p = _pad2(W4, P3, P4)
    b1p = jnp.pad(b1, (0, P1 - b1.shape[0])).reshape(1, P1)
    b2p = jnp.pad(b2, (0, P2 - b2.shape[0])).reshape(1, P2)
    b3p = jnp.pad(b3, (0, P3 - b3.shape[0])).reshape(1, P3)
    b4p = jnp.pad(b4, (0, P4 - b4.shape[0])).reshape(1, P4)

    deg2 = _deg_kernel(dst_r)
    degt = deg2.T[:N]

    yw1, dinv = _tc_pre(x, w1p, degt)
    acc1 = _make_agg(P1, *_CFG1)(yw1, s64, d64, jnp.zeros((_CFG1[1], P1), jnp.float32))
    yw2 = _tc_mid(acc1, yw1, dinv, b1p, w2p)
    acc2 = _make_agg(P2, *_CFG2)(yw2, s128, d128, jnp.zeros((_CFG2[1], P2), jnp.float32))
    yw3 = _tc_mid(acc2, yw2, dinv, b2p, w3p)
    acc3 = _make_agg(P3, *_CFG3)(yw3, s128, d128, jnp.zeros((_CFG3[1], P3), jnp.float32))
    yw4 = _tc_mid(acc3, yw3, dinv, b3p, w4p)
    acc4 = _make_agg(P4, *_CFG4)(yw4, s128, d128, jnp.zeros((_CFG4[1], P4), jnp.float32))
    out = _tc_post(acc4, yw4, dinv, b4p)
    return out[:, :1]


# speedup vs baseline: 2.0280x; 2.0280x over previous
import jax
import jax.numpy as jnp
from jax import lax
from jax.experimental import pallas as pl
from jax.experimental.pallas import tpu as pltpu
from jax.experimental.pallas import tpu_sc as plsc

N = 10000
E = 160000
NC, NS = 2, 16
KB = 128
NB = 40
EW = KB * NB
EP = EW * NC * NS
NROW = 10240
RPT = NROW // NS
DUMP = N

P1, P2, P3, P4 = 112, 64, 32, 16
RB = 1000


def _deg_body(dst_hbm, deg_out, dst_v, degtab, sumv, shared):
    cid = lax.axis_index("c")
    sid = lax.axis_index("s")
    pltpu.sync_copy(dst_hbm.at[cid, sid], dst_v)

    def _zero(i, _):
        degtab[pl.ds(i * 16, 16)] = jnp.zeros((16,), jnp.float32)
        return 0
    lax.fori_loop(0, NROW // 16, _zero, 0)

    ones = jnp.ones((16,), jnp.float32)

    def _hist(i, _):
        j = i // (KB // 16)
        c = i % (KB // 16)
        idx = dst_v[j, pl.ds(c * 16, 16)]
        plsc.addupdate_scatter(degtab, [idx], ones)
        return 0
    lax.fori_loop(0, EW // 16, _hist, 0)

    pltpu.sync_copy(degtab, shared.at[sid])
    plsc.subcore_barrier()
    pltpu.sync_copy(shared.at[:, pl.ds(sid * RPT, RPT)], sumv)

    def _red(ci, _):
        a = jnp.zeros((16,), jnp.float32)
        for r in range(NS):
            a = a + sumv[r, pl.ds(ci * 16, 16)]
        degtab[pl.ds(ci * 16, 16)] = a
        return 0
    lax.fori_loop(0, RPT // 16, _red, 0)
    pltpu.sync_copy(degtab.at[pl.ds(0, RPT)], deg_out.at[cid, pl.ds(sid * RPT, RPT)])


_deg_kernel = pl.kernel(
    _deg_body,
    out_type=jax.ShapeDtypeStruct((NC, NROW), jnp.float32),
    mesh=plsc.VectorSubcoreMesh(core_axis_name="c", subcore_axis_name="s"),
    compiler_params=pltpu.CompilerParams(needs_layout_passes=False),
    scratch_types=[
        pltpu.VMEM((NB, KB), jnp.int32),
        pltpu.VMEM((NROW,), jnp.float32),
        pltpu.VMEM((NS, RPT), jnp.float32),
        pltpu.VMEM_SHARED((NS, NROW), jnp.float32),
    ],
)


def _make_agg_body(nbuf, kb, d):
    nb = EP // NS // kb

    def _agg_body(yw_hbm, src_hbm, dst_hbm, zrows_hbm, acc_out,
                  src_v, dst_v, *rest):
        rows = rest[:nbuf]
        gsems = rest[nbuf:2 * nbuf]
        ssems = rest[2 * nbuf:3 * nbuf]
        acc_sp = rest[3 * nbuf]
        cid = lax.axis_index("c")
        sid = lax.axis_index("s")

        @pl.when(cid == 0)
        def _work():
            pltpu.sync_copy(zrows_hbm, rows[0])
            for r in range(RPT // kb):
                pltpu.sync_copy(rows[0], acc_sp.at[pl.ds(sid * RPT + r * kb, kb)])
            plsc.subcore_barrier()

            pltpu.sync_copy(src_hbm.at[sid], src_v)
            pltpu.sync_copy(dst_hbm.at[sid], dst_v)

            def _gather(i, b, sem):
                return pltpu.make_async_copy(yw_hbm.at[src_v.at[i]], rows[b], sem)

            def _scatter(i, b, sem):
                return pltpu.make_async_copy(rows[b], acc_sp.at[dst_v.at[i]], sem)

            for b in range(nbuf):
                _gather(b, b, gsems[b]).start()

            nw = jnp.where(cid == 0, nb // nbuf, 0)

            def _wave(w, _):
                i0 = w * nbuf
                for b in range(nbuf):
                    _gather(i0 + b, b, gsems[b]).wait()
                    _scatter(i0 + b, b, ssems[b]).start(add=True)
                for b in range(nbuf):
                    _scatter(i0 + b, b, ssems[b]).wait()
                    _gather(i0 + nbuf + b, b, gsems[b]).start()
                return 0
            lax.fori_loop(0, nw - 1, _wave, 0)
            i0 = (nw - 1) * nbuf
            for b in range(nbuf):
                _gather(i0 + b, b, gsems[b]).wait()
                _scatter(i0 + b, b, ssems[b]).start(add=True)
            for b in range(nbuf):
                _scatter(i0 + b, b, ssems[b]).wait()
            plsc.subcore_barrier()
            pltpu.sync_copy(acc_sp.at[pl.ds(sid * RPT, RPT)],
                            acc_out.at[pl.ds(sid * RPT, RPT)])
    return _agg_body


def _make_agg(d, nbuf, kb):
    nb = EP // NS // kb
    return pl.kernel(
        _make_agg_body(nbuf, kb, d),
        out_type=jax.ShapeDtypeStruct((NROW, d), jnp.float32),
        mesh=plsc.VectorSubcoreMesh(core_axis_name="c", subcore_axis_name="s"),
        compiler_params=pltpu.CompilerParams(use_tc_tiling_on_sc=False),
        scratch_types=(
            [pltpu.VMEM((nb, kb), jnp.int32),
             pltpu.VMEM((nb, kb), jnp.int32)]
            + [pltpu.VMEM((kb, d), jnp.float32) for _ in range(nbuf)]
            + [pltpu.SemaphoreType.DMA for _ in range(2 * nbuf)]
            + [pltpu.VMEM_SHARED((NROW, d), jnp.float32)]
        ),
    )


def _pre_body(x_ref, w_ref, degt_ref, yw_ref, dinv_ref):
    deg = degt_ref[:, 0:1] + degt_ref[:, 1:2] + 1.0
    dv = lax.rsqrt(deg)
    xw = jnp.dot(x_ref[...], w_ref[...], preferred_element_type=jnp.float32)
    yw_ref[...] = dv * xw
    dinv_ref[...] = dv


def _tc_pre(x, w1p, degt):
    return pl.pallas_call(
        _pre_body,
        grid=(N // RB,),
        in_specs=[
            pl.BlockSpec((RB, x.shape[1]), lambda i: (i, 0)),
            pl.BlockSpec((w1p.shape[0], w1p.shape[1]), lambda i: (0, 0)),
            pl.BlockSpec((RB, 2), lambda i: (i, 0)),
        ],
        out_specs=[
            pl.BlockSpec((RB, w1p.shape[1]), lambda i: (i, 0)),
            pl.BlockSpec((RB, 1), lambda i: (i, 0)),
        ],
        out_shape=[
            jax.ShapeDtypeStruct((N, w1p.shape[1]), jnp.float32),
            jax.ShapeDtypeStruct((N, 1), jnp.float32),
        ],
    )(x, w1p, degt)


def _mid_body(acc_ref, yw_ref, dinv_ref, b_ref, w_ref, out_ref):
    dv = dinv_ref[...]
    h = dv * (acc_ref[...] + yw_ref[...]) + b_ref[...]
    h = jnp.maximum(h, 0.0)
    out_ref[...] = dv * jnp.dot(h, w_ref[...], preferred_element_type=jnp.float32)


def _tc_mid(acc, yw, dinv, bp, wp):
    din, dout = wp.shape
    return pl.pallas_call(
        _mid_body,
        grid=(N // RB,),
        in_specs=[
            pl.BlockSpec((RB, din), lambda i: (i, 0)),
            pl.BlockSpec((RB, din), lambda i: (i, 0)),
            pl.BlockSpec((RB, 1), lambda i: (i, 0)),
            pl.BlockSpec((1, din), lambda i: (0, 0)),
            pl.BlockSpec((din, dout), lambda i: (0, 0)),
        ],
        out_specs=pl.BlockSpec((RB, dout), lambda i: (i, 0)),
        out_shape=jax.ShapeDtypeStruct((N, dout), jnp.float32),
    )(acc, yw, dinv, bp, wp)


def _post_body(acc_ref, yw_ref, dinv_ref, b_ref, out_ref):
    dv = dinv_ref[...]
    out_ref[...] = dv * (acc_ref[...] + yw_ref[...]) + b_ref[...]


def _tc_post(acc, yw, dinv, bp):
    din = yw.shape[1]
    return pl.pallas_call(
        _post_body,
        grid=(N // RB,),
        in_specs=[
            pl.BlockSpec((RB, din), lambda i: (i, 0)),
            pl.BlockSpec((RB, din), lambda i: (i, 0)),
            pl.BlockSpec((RB, 1), lambda i: (i, 0)),
            pl.BlockSpec((1, din), lambda i: (0, 0)),
        ],
        out_specs=pl.BlockSpec((RB, din), lambda i: (i, 0)),
        out_shape=jax.ShapeDtypeStruct((N, din), jnp.float32),
    )(acc, yw, dinv, bp)


def _pad2(a, rows, cols):
    return jnp.pad(a, ((0, rows - a.shape[0]), (0, cols - a.shape[1])))


_CFG1 = (4, 64)
_CFG2 = (8, 128)
_CFG3 = (8, 128)
_CFG4 = (8, 128)


def kernel(x, edge_index, W1, b1, W2, b2, W3, b3, W4, b4):
    src = edge_index[0].astype(jnp.int32)
    dst = edge_index[1].astype(jnp.int32)
    pad = EP - E
    ar = jnp.arange(pad, dtype=jnp.int32)
    srcp = jnp.concatenate([src, ar % N])
    dstp = jnp.concatenate([dst, DUMP + (ar % (NROW - N))])
    src_r = srcp.reshape(NC, NS, NB, KB)
    dst_r = dstp.reshape(NC, NS, NB, KB)
    s64 = srcp.reshape(NS, EP // NS // 64, 64)
    d64 = dstp.reshape(NS, EP // NS // 64, 64)
    s128 = srcp.reshape(NS, EP // NS // 128, 128)
    d128 = dstp.reshape(NS, EP // NS // 128, 128)

    w1p = _pad2(W1, 256, P1)
    w2p = _pad2(W2, P1, P2)
    w3p = _pad2(W3, P2, P3)
    w4p = _pad2(W4, P3, P4)
    b1p = jnp.pad(b1, (0, P1 - b1.shape[0])).reshape(1, P1)
    b2p = jnp.pad(b2, (0, P2 - b2.shape[0])).reshape(1, P2)
    b3p = jnp.pad(b3, (0, P3 - b3.shape[0])).reshape(1, P3)
    b4p = jnp.pad(b4, (0, P4 - b4.shape[0])).reshape(1, P4)

    deg2 = _deg_kernel(dst_r)
    degt = deg2.T[:N]

    yw1, dinv = _tc_pre(x, w1p, degt)
    acc1 = _make_agg(P1, *_CFG1)(yw1, s64, d64, jnp.zeros((_CFG1[1], P1), jnp.float32))
    yw2 = _tc_mid(acc1, yw1, dinv, b1p, w2p)
    acc2 = _make_agg(P2, *_CFG2)(yw2, s128, d128, jnp.zeros((_CFG2[1], P2), jnp.float32))
    yw3 = _tc_mid(acc2, yw2, dinv, b2p, w3p)
    acc3 = _make_agg(P3, *_CFG3)(yw3, s128, d128, jnp.zeros((_CFG3[1], P3), jnp.float32))
    yw4 = _tc_mid(acc3, yw3, dinv, b3p, w4p)
    acc4 = _make_agg(P4, *_CFG4)(yw4, s128, d128, jnp.zeros((_CFG4[1], P4), jnp.float32))
    out = _tc_post(acc4, yw4, dinv, b4p)
    return out[:, :1]
